# Initial kernel scaffold; baseline (speedup 1.0000x reference)
import jax
import jax.numpy as jnp
from jax.experimental import pallas as pl


def kernel(node_features, pos, edge_index, batch, node_atom, Wm_l, Wr_l, Ws_l, Wo_l, w_score, Wm_g, Wr_g, Ws_g, Wo_g, W_unpool):
    raise NotImplementedError("write your pallas kernel here")


# speedup vs baseline: 1.1287x; 1.1287x over previous
import functools
import math

import jax
import jax.numpy as jnp
from jax import lax
from jax.experimental import pallas as pl
from jax.experimental.pallas import tpu as pltpu

N = 10000
E = 320000
D = 128
B = 16
NPG = N // B
K_MASTERS = 32
NUM_RBF = 32
CUTOFF = 5.0
SQRT3 = math.sqrt(3.0)
RBF_STEP = CUTOFF / (NUM_RBF - 1)
RBF_COEFF = -0.5 / (CUTOFF / NUM_RBF) ** 2


_BN = 2000


def _matmul_body(x_ref, w_ref, o_ref):
    o_ref[...] = jnp.dot(x_ref[...], w_ref[...],
                         preferred_element_type=jnp.float32)


def _node_matmul(x, w):
    return pl.pallas_call(
        _matmul_body,
        grid=(N // _BN,),
        in_specs=[
            pl.BlockSpec((_BN, D), lambda i: (i, 0)),
            pl.BlockSpec((D, D), lambda i: (0, 0)),
        ],
        out_specs=pl.BlockSpec((_BN, D), lambda i: (i, 0)),
        out_shape=jax.ShapeDtypeStruct((N, D), jnp.float32),
    )(x, w)


_BE = 3200


def _edge_msg_body(xmsrc_ref, ev_ref, wr_ref, ws_ref, offs_ref, o_ref):
    ev = ev_ref[...]
    n2 = jnp.sum(ev * ev, axis=1, keepdims=True)
    n = jnp.sqrt(n2 + 1e-12)
    vhat = ev / n
    sh = jnp.concatenate([jnp.ones_like(n), SQRT3 * vhat[:, :3]], axis=1)
    diff = n - offs_ref[...]
    rbf = jnp.exp(RBF_COEFF * (diff * diff))
    pre = (xmsrc_ref[...]
           + jnp.dot(rbf, wr_ref[...], preferred_element_type=jnp.float32)
           + jnp.dot(sh, ws_ref[...], preferred_element_type=jnp.float32))
    o_ref[...] = pre * jax.nn.sigmoid(pre)


def _edge_messages(xmsrc, ev4, wr, ws):
    offs = jnp.linspace(0.0, CUTOFF, NUM_RBF).reshape(1, NUM_RBF)
    return pl.pallas_call(
        _edge_msg_body,
        grid=(E // _BE,),
        in_specs=[
            pl.BlockSpec((_BE, D), lambda i: (i, 0)),
            pl.BlockSpec((_BE, 4), lambda i: (i, 0)),
            pl.BlockSpec((NUM_RBF, D), lambda i: (0, 0)),
            pl.BlockSpec((4, D), lambda i: (0, 0)),
            pl.BlockSpec((1, NUM_RBF), lambda i: (0, 0)),
        ],
        out_specs=pl.BlockSpec((_BE, D), lambda i: (i, 0)),
        out_shape=jax.ShapeDtypeStruct((E, D), jnp.float32),
    )(xmsrc, ev4, wr, ws, offs)


def _out_proj_body(agg_ref, x_ref, wo_ref, h_ref):
    h_ref[...] = jnp.dot(agg_ref[...], wo_ref[...],
                         preferred_element_type=jnp.float32) + x_ref[...]


def _out_proj(agg, x, wo):
    return pl.pallas_call(
        _out_proj_body,
        grid=(N // _BN,),
        in_specs=[
            pl.BlockSpec((_BN, D), lambda i: (i, 0)),
            pl.BlockSpec((_BN, D), lambda i: (i, 0)),
            pl.BlockSpec((D, D), lambda i: (0, 0)),
        ],
        out_specs=pl.BlockSpec((_BN, D), lambda i: (i, 0)),
        out_shape=jax.ShapeDtypeStruct((N, D), jnp.float32),
    )(agg, x, wo)


_K = K_MASTERS


_KK = K_MASTERS * K_MASTERS


def _master_proj_body(agg_ref, hm_ref, wo_ref, wu_ref, o_ref):
    hg = (jnp.dot(agg_ref[...], wo_ref[...],
                  preferred_element_type=jnp.float32) + hm_ref[...])
    o_ref[...] = jnp.dot(hg, wu_ref[...], preferred_element_type=jnp.float32)


def _master_proj(agg, h_m, wo, wu):
    nm = B * _K
    return pl.pallas_call(
        _master_proj_body,
        grid=(1,),
        in_specs=[
            pl.BlockSpec((nm, D), lambda i: (0, 0)),
            pl.BlockSpec((nm, D), lambda i: (0, 0)),
            pl.BlockSpec((D, D), lambda i: (0, 0)),
            pl.BlockSpec((D, D), lambda i: (0, 0)),
        ],
        out_specs=pl.BlockSpec((nm, D), lambda i: (0, 0)),
        out_shape=jax.ShapeDtypeStruct((nm, D), jnp.float32),
    )(agg, h_m, wo, wu)


def _global_mp_body(hm_ref, pm_ref, sel_s_ref, sel_d_ref, sel_dt_ref,
                    mask_ref, wm_ref, wr_ref, wsv_ref, ws0_ref, o_ref):
    hm = hm_ref[...]
    pm = pm_ref[...]
    sel_s = sel_s_ref[...]
    sel_d = sel_d_ref[...]
    xmg = jnp.dot(hm, wm_ref[...], preferred_element_type=jnp.float32)
    ps = jnp.dot(sel_s, pm, preferred_element_type=jnp.float32)
    pd = jnp.dot(sel_d, pm, preferred_element_type=jnp.float32)
    ev = ps - pd
    n2 = jnp.sum(ev * ev, axis=1, keepdims=True)
    n = jnp.sqrt(n2 + 1e-12)
    shc = (jnp.dot(ev, wsv_ref[...], preferred_element_type=jnp.float32)
           * (SQRT3 / n) + ws0_ref[...])
    offs = lax.broadcasted_iota(
        jnp.int32, (1, NUM_RBF), 1).astype(jnp.float32) * RBF_STEP
    diff = n - offs
    rbf = jnp.exp(RBF_COEFF * diff * diff)
    pre = (jnp.dot(sel_s, xmg, preferred_element_type=jnp.float32)
           + jnp.dot(rbf, wr_ref[...], preferred_element_type=jnp.float32)
           + shc)
    msg = pre * jax.nn.sigmoid(pre) * mask_ref[...]
    o_ref[...] = jnp.dot(sel_dt_ref[...], msg,
                         preferred_element_type=jnp.float32)


def _global_mp(h_m, pm4, wm, wr, ws):
    k = K_MASTERS
    ii = jnp.arange(_KK, dtype=jnp.int32)
    s_of = ii // k
    d_of = ii % k
    sel_s = jax.nn.one_hot(s_of, k, dtype=jnp.float32)
    sel_d = jax.nn.one_hot(d_of, k, dtype=jnp.float32)
    sel_dt = sel_d.T
    mask = (s_of != d_of).astype(jnp.float32).reshape(_KK, 1)
    wsv = jnp.concatenate([ws[1:4], jnp.zeros((1, D), jnp.float32)], axis=0)
    ws0 = ws[0:1]
    return pl.pallas_call(
        _global_mp_body,
        grid=(B,),
        in_specs=[
            pl.BlockSpec((_K, D), lambda g: (g, 0)),
            pl.BlockSpec((_K, 4), lambda g: (g, 0)),
            pl.BlockSpec((_KK, _K), lambda g: (0, 0)),
            pl.BlockSpec((_KK, _K), lambda g: (0, 0)),
            pl.BlockSpec((_K, _KK), lambda g: (0, 0)),
            pl.BlockSpec((_KK, 1), lambda g: (0, 0)),
            pl.BlockSpec((D, D), lambda g: (0, 0)),
            pl.BlockSpec((NUM_RBF, D), lambda g: (0, 0)),
            pl.BlockSpec((4, D), lambda g: (0, 0)),
            pl.BlockSpec((1, D), lambda g: (0, 0)),
        ],
        out_specs=pl.BlockSpec((_K, D), lambda g: (g, 0)),
        out_shape=jax.ShapeDtypeStruct((B * _K, D), jnp.float32),
    )(h_m, pm4, sel_s, sel_d, sel_dt, mask, wm, wr, wsv, ws0)


def kernel(node_features, pos, edge_index, batch, node_atom,
           Wm_l, Wr_l, Ws_l, Wo_l, w_score,
           Wm_g, Wr_g, Ws_g, Wo_g, W_unpool):
    src, dst = edge_index[0], edge_index[1]
    pos4 = jnp.pad(pos, ((0, 0), (0, 1)))

    xm = _node_matmul(node_features, Wm_l)
    ev4 = pos4[src] - pos4[dst]
    xmsrc = xm[src]
    msg = _edge_messages(xmsrc, ev4, Wr_l, Ws_l)
    agg = jax.ops.segment_sum(msg, dst, num_segments=N)
    h_local = _out_proj(agg, node_features, Wo_l)
    scores = h_local @ w_score

    _, topk_idx = lax.top_k(scores.reshape(B, NPG), K_MASTERS)
    topk_idx = jnp.sort(topk_idx, axis=1)
    master_idx = (topk_idx + (jnp.arange(B) * NPG)[:, None]).reshape(-1)

    h_m = h_local[master_idx]
    pm4 = pos4[master_idx]
    agg_g = _global_mp(h_m, pm4, Wm_g, Wr_g, Ws_g)
    h_update = _master_proj(agg_g, h_m, Wo_g, W_unpool)
    h_out = h_local.at[master_idx].add(h_update)
    return (h_out, pos, edge_index, batch, node_atom)


# speedup vs baseline: 1.7448x; 1.5458x over previous
import functools
import math

import jax
import jax.numpy as jnp
from jax import lax
from jax.experimental import pallas as pl
from jax.experimental.pallas import tpu as pltpu
from jax.experimental.pallas import tpu_sc as plsc

N = 10000
E = 320000
D = 128
B = 16
NPG = N // B
K_MASTERS = 32
NUM_RBF = 32
CUTOFF = 5.0
SQRT3 = math.sqrt(3.0)
RBF_STEP = CUTOFF / (NUM_RBF - 1)
RBF_COEFF = -0.5 / (CUTOFF / NUM_RBF) ** 2


_BN = 2000


def _matmul_body(x_ref, w_ref, o_ref):
    o_ref[...] = jnp.dot(x_ref[...], w_ref[...],
                         preferred_element_type=jnp.float32)


def _node_matmul(x, w):
    return pl.pallas_call(
        _matmul_body,
        grid=(N // _BN,),
        in_specs=[
            pl.BlockSpec((_BN, D), lambda i: (i, 0)),
            pl.BlockSpec((D, D), lambda i: (0, 0)),
        ],
        out_specs=pl.BlockSpec((_BN, D), lambda i: (i, 0)),
        out_shape=jax.ShapeDtypeStruct((N, D), jnp.float32),
    )(x, w)


_BE = 3200


def _edge_geom_body(ev_ref, wr_ref, ws_ref, offs_ref, o_ref):
    ev = ev_ref[...]
    n2 = jnp.sum(ev * ev, axis=1, keepdims=True)
    n = jnp.sqrt(n2 + 1e-12)
    vhat = ev / n
    sh = jnp.concatenate([jnp.ones_like(n), SQRT3 * vhat[:, :3]], axis=1)
    diff = n - offs_ref[...]
    rbf = jnp.exp(RBF_COEFF * (diff * diff))
    o_ref[...] = (
        jnp.dot(rbf, wr_ref[...], preferred_element_type=jnp.float32)
        + jnp.dot(sh, ws_ref[...], preferred_element_type=jnp.float32))


def _edge_geom(ev4, wr, ws):
    offs = jnp.linspace(0.0, CUTOFF, NUM_RBF).reshape(1, NUM_RBF)
    return pl.pallas_call(
        _edge_geom_body,
        grid=(E // _BE,),
        in_specs=[
            pl.BlockSpec((_BE, 4), lambda i: (i, 0)),
            pl.BlockSpec((NUM_RBF, D), lambda i: (0, 0)),
            pl.BlockSpec((4, D), lambda i: (0, 0)),
            pl.BlockSpec((1, NUM_RBF), lambda i: (0, 0)),
        ],
        out_specs=pl.BlockSpec((_BE, D), lambda i: (i, 0)),
        out_shape=jax.ShapeDtypeStruct((E, D), jnp.float32),
    )(ev4, wr, ws, offs)


def _out_proj_body(agg0_ref, agg1_ref, x_ref, wo_ref, h_ref):
    agg = agg0_ref[...] + agg1_ref[...]
    h_ref[...] = jnp.dot(agg, wo_ref[...],
                         preferred_element_type=jnp.float32) + x_ref[...]


def _out_proj(agg0, agg1, x, wo):
    return pl.pallas_call(
        _out_proj_body,
        grid=(N // _BN,),
        in_specs=[
            pl.BlockSpec((_BN, D), lambda i: (i, 0)),
            pl.BlockSpec((_BN, D), lambda i: (i, 0)),
            pl.BlockSpec((_BN, D), lambda i: (i, 0)),
            pl.BlockSpec((D, D), lambda i: (0, 0)),
        ],
        out_specs=pl.BlockSpec((_BN, D), lambda i: (i, 0)),
        out_shape=jax.ShapeDtypeStruct((N, D), jnp.float32),
    )(agg0, agg1, x, wo)


_K = K_MASTERS


_KK = K_MASTERS * K_MASTERS


def _master_proj_body(agg_ref, hm_ref, wo_ref, wu_ref, o_ref):
    hg = (jnp.dot(agg_ref[...], wo_ref[...],
                  preferred_element_type=jnp.float32) + hm_ref[...])
    o_ref[...] = jnp.dot(hg, wu_ref[...], preferred_element_type=jnp.float32)


def _master_proj(agg, h_m, wo, wu):
    nm = B * _K
    return pl.pallas_call(
        _master_proj_body,
        grid=(1,),
        in_specs=[
            pl.BlockSpec((nm, D), lambda i: (0, 0)),
            pl.BlockSpec((nm, D), lambda i: (0, 0)),
            pl.BlockSpec((D, D), lambda i: (0, 0)),
            pl.BlockSpec((D, D), lambda i: (0, 0)),
        ],
        out_specs=pl.BlockSpec((nm, D), lambda i: (0, 0)),
        out_shape=jax.ShapeDtypeStruct((nm, D), jnp.float32),
    )(agg, h_m, wo, wu)


def _global_mp_body(hm_ref, pm_ref, sel_s_ref, sel_d_ref, sel_dt_ref,
                    mask_ref, wm_ref, wr_ref, wsv_ref, ws0_ref, o_ref):
    hm = hm_ref[...]
    pm = pm_ref[...]
    sel_s = sel_s_ref[...]
    sel_d = sel_d_ref[...]
    xmg = jnp.dot(hm, wm_ref[...], preferred_element_type=jnp.float32)
    ps = jnp.dot(sel_s, pm, preferred_element_type=jnp.float32)
    pd = jnp.dot(sel_d, pm, preferred_element_type=jnp.float32)
    ev = ps - pd
    n2 = jnp.sum(ev * ev, axis=1, keepdims=True)
    n = jnp.sqrt(n2 + 1e-12)
    shc = (jnp.dot(ev, wsv_ref[...], preferred_element_type=jnp.float32)
           * (SQRT3 / n) + ws0_ref[...])
    offs = lax.broadcasted_iota(
        jnp.int32, (1, NUM_RBF), 1).astype(jnp.float32) * RBF_STEP
    diff = n - offs
    rbf = jnp.exp(RBF_COEFF * diff * diff)
    pre = (jnp.dot(sel_s, xmg, preferred_element_type=jnp.float32)
           + jnp.dot(rbf, wr_ref[...], preferred_element_type=jnp.float32)
           + shc)
    msg = pre * jax.nn.sigmoid(pre) * mask_ref[...]
    o_ref[...] = jnp.dot(sel_dt_ref[...], msg,
                         preferred_element_type=jnp.float32)


def _global_mp(h_m, pm4, wm, wr, ws):
    k = K_MASTERS
    ii = jnp.arange(_KK, dtype=jnp.int32)
    s_of = ii // k
    d_of = ii % k
    sel_s = jax.nn.one_hot(s_of, k, dtype=jnp.float32)
    sel_d = jax.nn.one_hot(d_of, k, dtype=jnp.float32)
    sel_dt = sel_d.T
    mask = (s_of != d_of).astype(jnp.float32).reshape(_KK, 1)
    wsv = jnp.concatenate([ws[1:4], jnp.zeros((1, D), jnp.float32)], axis=0)
    ws0 = ws[0:1]
    return pl.pallas_call(
        _global_mp_body,
        grid=(B,),
        in_specs=[
            pl.BlockSpec((_K, D), lambda g: (g, 0)),
            pl.BlockSpec((_K, 4), lambda g: (g, 0)),
            pl.BlockSpec((_KK, _K), lambda g: (0, 0)),
            pl.BlockSpec((_KK, _K), lambda g: (0, 0)),
            pl.BlockSpec((_K, _KK), lambda g: (0, 0)),
            pl.BlockSpec((_KK, 1), lambda g: (0, 0)),
            pl.BlockSpec((D, D), lambda g: (0, 0)),
            pl.BlockSpec((NUM_RBF, D), lambda g: (0, 0)),
            pl.BlockSpec((4, D), lambda g: (0, 0)),
            pl.BlockSpec((1, D), lambda g: (0, 0)),
        ],
        out_specs=pl.BlockSpec((_K, D), lambda g: (g, 0)),
        out_shape=jax.ShapeDtypeStruct((B * _K, D), jnp.float32),
    )(h_m, pm4, sel_s, sel_d, sel_dt, mask, wm, wr, wsv, ws0)


_EC = 80
_NCHUNK = 125
_NSUB = 16
_ROWS_PER_SUB = 632
_NPAD = _NSUB * _ROWS_PER_SUB


def _scatter_body(xm_hbm, geom_hbm, src_hbm, dst_hbm, zero_hbm, out_hbm,
                  si0, si1, di0, di1, g0, g1, mbuf, agg_sh,
                  ssi0, ssi1, sdi0, sdi1, sg0, sg1, sm):
    c = lax.axis_index("c")
    s = lax.axis_index("s")
    w = c * _NSUB + s
    base = w * (_EC * _NCHUNK)

    pltpu.sync_copy(zero_hbm.at[pl.ds(s * _ROWS_PER_SUB, _ROWS_PER_SUB)],
                    agg_sh.at[pl.ds(s * _ROWS_PER_SUB, _ROWS_PER_SUB)])
    plsc.subcore_barrier()

    sib = (si0, si1)
    dib = (di0, di1)
    gb = (g0, g1)
    ssib = (ssi0, ssi1)
    sdib = (sdi0, sdi1)
    sgb = (sg0, sg1)

    def fire_idx(j, b):
        pltpu.async_copy(src_hbm.at[w, j], sib[b], ssib[b])
        pltpu.async_copy(dst_hbm.at[w, j], dib[b], sdib[b])

    def wait_idx(j, b):
        pltpu.make_async_copy(src_hbm.at[w, j], sib[b], ssib[b]).wait()
        pltpu.make_async_copy(dst_hbm.at[w, j], dib[b], sdib[b]).wait()

    def fire_gather(j, b):
        pltpu.async_copy(xm_hbm.at[sib[b]], gb[b], sgb[b])

    def wait_gather(j, b):
        pltpu.make_async_copy(xm_hbm.at[sib[b]], gb[b], sgb[b]).wait()

    def fire_geom(j):
        pltpu.async_copy(geom_hbm.at[pl.ds(base + j * _EC, _EC)], mbuf, sm)

    def wait_geom(j):
        pltpu.make_async_copy(geom_hbm.at[pl.ds(base + j * _EC, _EC)],
                              mbuf, sm).wait()

    def compute(b):
        def row(r):
            for k in range(D // 16):
                sl = pl.ds(k * 16, 16)
                v = gb[b][r, sl] + mbuf[r, sl]
                mbuf[r, sl] = v / (1.0 + jnp.exp(-v))

        pl.loop(0, _EC)(row)

    def scatter(b):
        pltpu.sync_copy(mbuf, agg_sh.at[dib[b]], add=True)

    fire_idx(0, 0)
    wait_idx(0, 0)
    fire_gather(0, 0)
    fire_geom(0)
    fire_idx(1, 1)

    def half(j, b):
        wait_gather(j, b)
        wait_geom(j)
        compute(b)
        wait_idx(j + 1, 1 - b)
        fire_gather(j + 1, 1 - b)
        scatter(b)
        fire_geom(j + 1)
        fire_idx(j + 2, b)

    def step(jj):
        j = 2 * jj
        half(j, 0)
        half(j + 1, 1)

    pl.loop(0, (_NCHUNK - 1) // 2)(step)
    wait_idx(_NCHUNK, 1)
    wait_gather(_NCHUNK - 1, 0)
    wait_geom(_NCHUNK - 1)
    compute(0)
    scatter(0)

    plsc.subcore_barrier()
    pltpu.sync_copy(agg_sh.at[pl.ds(s * _ROWS_PER_SUB, _ROWS_PER_SUB)],
                    out_hbm.at[c, pl.ds(s * _ROWS_PER_SUB, _ROWS_PER_SUB)])


def _sc_gather_silu_scatter(xm, geom, src, dst):
    src3d = jnp.pad(src.reshape(2 * _NSUB, _NCHUNK, _EC),
                    ((0, 0), (0, 1), (0, 0)))
    dst3d = jnp.pad(dst.reshape(2 * _NSUB, _NCHUNK, _EC),
                    ((0, 0), (0, 1), (0, 0)))
    zeros = jnp.zeros((_NPAD, D), jnp.float32)
    mesh = plsc.VectorSubcoreMesh(core_axis_name="c", subcore_axis_name="s")
    f = pl.kernel(
        _scatter_body,
        out_type=jax.ShapeDtypeStruct((2, _NPAD, D), jnp.float32),
        mesh=mesh,
        scratch_types=[
            pltpu.VMEM((_EC,), jnp.int32),
            pltpu.VMEM((_EC,), jnp.int32),
            pltpu.VMEM((_EC,), jnp.int32),
            pltpu.VMEM((_EC,), jnp.int32),
            pltpu.VMEM((_EC, D), jnp.float32),
            pltpu.VMEM((_EC, D), jnp.float32),
            pltpu.VMEM((_EC, D), jnp.float32),
            pltpu.VMEM_SHARED((_NPAD, D), jnp.float32),
            pltpu.SemaphoreType.DMA,
            pltpu.SemaphoreType.DMA,
            pltpu.SemaphoreType.DMA,
            pltpu.SemaphoreType.DMA,
            pltpu.SemaphoreType.DMA,
            pltpu.SemaphoreType.DMA,
            pltpu.SemaphoreType.DMA,
        ],
    )
    parts = f(xm, geom, src3d, dst3d, zeros)
    return parts[:, :N]


def kernel(node_features, pos, edge_index, batch, node_atom,
           Wm_l, Wr_l, Ws_l, Wo_l, w_score,
           Wm_g, Wr_g, Ws_g, Wo_g, W_unpool):
    src, dst = edge_index[0], edge_index[1]
    pos4 = jnp.pad(pos, ((0, 0), (0, 1)))

    xm = _node_matmul(node_features, Wm_l)
    ev4 = pos4[src] - pos4[dst]
    geom = _edge_geom(ev4, Wr_l, Ws_l)
    parts = _sc_gather_silu_scatter(xm, geom, src, dst)
    h_local = _out_proj(parts[0], parts[1], node_features, Wo_l)
    scores = h_local @ w_score

    _, topk_idx = lax.top_k(scores.reshape(B, NPG), K_MASTERS)
    topk_idx = jnp.sort(topk_idx, axis=1)
    master_idx = (topk_idx + (jnp.arange(B) * NPG)[:, None]).reshape(-1)

    h_m = h_local[master_idx]
    pm4 = pos4[master_idx]
    agg_g = _global_mp(h_m, pm4, Wm_g, Wr_g, Ws_g)
    h_update = _master_proj(agg_g, h_m, Wo_g, W_unpool)
    h_out = h_local.at[master_idx].add(h_update)
    return (h_out, pos, edge_index, batch, node_atom)
